# baseline (speedup 1.0000x reference)
import functools

import jax
import jax.numpy as jnp
from jax import lax
from jax.experimental import pallas as pl
from jax.experimental.pallas import tpu as pltpu
from jax.experimental.pallas import tpu_sc as plsc


_DNUM_T_LHS = (((0,), (0,)), ((), ()))


def _tables_body(nst_ref, w1a_ref, w1b_ref, b1_ref, gs_ref, gd_ref):
    nst = nst_ref[...]
    gs = (
        lax.dot_general(nst, w1a_ref[...], _DNUM_T_LHS,
                        preferred_element_type=jnp.float32)
        + b1_ref[...]
    )
    gd = lax.dot_general(nst, w1b_ref[...], _DNUM_T_LHS,
                         preferred_element_type=jnp.float32)
    gs_ref[...] = gs
    gd_ref[...] = gd


def _make_tables(ns_t, w1a, w1b, b1_row):
    s, n = ns_t.shape
    f = w1a.shape[1]
    tn = 2048
    grid = (n + tn - 1) // tn
    return pl.pallas_call(
        _tables_body,
        grid=(grid,),
        in_specs=[
            pl.BlockSpec((s, tn), lambda i: (0, i)),
            pl.BlockSpec((s, f), lambda i: (0, 0)),
            pl.BlockSpec((s, f), lambda i: (0, 0)),
            pl.BlockSpec((1, f), lambda i: (0, 0)),
        ],
        out_specs=[
            pl.BlockSpec((tn, f), lambda i: (i, 0)),
            pl.BlockSpec((tn, f), lambda i: (i, 0)),
        ],
        out_shape=[
            jax.ShapeDtypeStruct((n, f), jnp.float32),
            jax.ShapeDtypeStruct((n, f), jnp.float32),
        ],
    )(ns_t, w1a, w1b, b1_row)


_C = 128
_TE = 6400


def _gather_add(gs, gd, edge_index):
    e = edge_index.shape[1]
    f = gs.shape[1]
    info = plsc.get_sparse_core_info()
    nw = info.num_cores * info.num_subcores
    n_chunks = e // _C
    assert e % _C == 0
    base_cnt = n_chunks // nw
    extra = n_chunks % nw
    maxc = base_cnt + (1 if extra else 0)
    mesh = plsc.VectorSubcoreMesh(core_axis_name="c", subcore_axis_name="s")

    @functools.partial(
        pl.kernel,
        mesh=mesh,
        compiler_params=pltpu.CompilerParams(use_tc_tiling_on_sc=False),
        out_type=jax.ShapeDtypeStruct((e // 2, 2 * f), jnp.float32),
        scratch_types=[
            pltpu.VMEM((maxc * _C,), jnp.int32),
            pltpu.VMEM((maxc * _C,), jnp.int32),
            pltpu.VMEM((_C, f), jnp.float32),
            pltpu.VMEM((_C, f), jnp.float32),
            pltpu.VMEM((_C, f), jnp.float32),
            pltpu.VMEM((_C, f), jnp.float32),
            pltpu.VMEM((_C, f), jnp.float32),
            pltpu.VMEM((_C, f), jnp.float32),
            pltpu.SemaphoreType.DMA,
            pltpu.SemaphoreType.DMA,
            pltpu.SemaphoreType.DMA,
            pltpu.SemaphoreType.DMA,
            pltpu.SemaphoreType.DMA,
            pltpu.SemaphoreType.DMA,
        ],
    )
    def k(gs_hbm, gd_hbm, ei_hbm, out_hbm,
          ixs, ixd, rs_a, rd_a, rs_b, rd_b, rs_c, rd_c,
          sem_a, sem_b, sem_c, sem_oa, sem_ob, sem_oc):
        wid = lax.axis_index("s") * info.num_cores + lax.axis_index("c")
        count = base_cnt + jnp.where(wid < extra, 1, 0)
        base_edge = (wid * base_cnt + jnp.minimum(wid, extra)) * _C
        nbase = base_cnt * _C
        pltpu.sync_copy(ei_hbm.at[0, pl.ds(base_edge, nbase)],
                        ixs.at[pl.ds(0, nbase)])
        pltpu.sync_copy(ei_hbm.at[1, pl.ds(base_edge, nbase)],
                        ixd.at[pl.ds(0, nbase)])

        @pl.when(count > base_cnt)
        def _():
            pltpu.sync_copy(ei_hbm.at[0, pl.ds(base_edge + nbase, _C)],
                            ixs.at[pl.ds(nbase, _C)])
            pltpu.sync_copy(ei_hbm.at[1, pl.ds(base_edge + nbase, _C)],
                            ixd.at[pl.ds(nbase, _C)])

        def fire(i, rs, rd, sem):
            o = i * _C
            pltpu.async_copy(gs_hbm.at[ixs.at[pl.ds(o, _C)]], rs, sem)
            pltpu.async_copy(gd_hbm.at[ixd.at[pl.ds(o, _C)]], rd, sem)

        def drain(rs, rd, sem):
            pltpu.make_async_copy(gs_hbm.at[pl.ds(0, _C)], rs, sem).wait()
            pltpu.make_async_copy(gs_hbm.at[pl.ds(0, _C)], rd, sem).wait()

        def process(i, rs, rd, sem_o):
            def row_body(rr, c2):
                for j in range(f // 16):
                    sl = pl.ds(j * 16, 16)
                    plsc.addupdate(rs.at[rr, sl], rd[rr, sl])
                return c2

            lax.fori_loop(0, _C, row_body, 0, unroll=2)
            eoff = base_edge + i * _C
            blk = eoff // _TE
            loc = eoff - blk * _TE
            half = loc // (_TE // 2)
            q0 = blk * (_TE // 2) + (loc - half * (_TE // 2))
            @pl.when(half == 0)
            def _():
                pltpu.async_copy(rs, out_hbm.at[pl.ds(q0, _C), pl.ds(0, f)],
                                 sem_o)

            @pl.when(half == 1)
            def _():
                pltpu.async_copy(rs, out_hbm.at[pl.ds(q0, _C), pl.ds(f, f)],
                                 sem_o)

        def wait_out(rs, sem_o):
            pltpu.make_async_copy(
                rs, out_hbm.at[pl.ds(0, _C), pl.ds(0, f)], sem_o).wait()

        bufs = ((rs_a, rd_a, sem_a, sem_oa),
                (rs_b, rd_b, sem_b, sem_ob),
                (rs_c, rd_c, sem_c, sem_oc))
        fire(0, rs_a, rd_a, sem_a)

        @pl.when(1 < count)
        def _():
            fire(1, rs_b, rd_b, sem_b)

        @pl.when(2 < count)
        def _():
            fire(2, rs_c, rd_c, sem_c)

        n_triples = (count + 2) // 3

        def triple_body(j, carry):
            i0 = 3 * j
            for t in range(3):
                it = i0 + t
                rs, rd, sem, sem_o = bufs[t]
                if t == 0:
                    drain(rs, rd, sem)
                    process(it, rs, rd, sem_o)
                else:
                    @pl.when(it < count)
                    def _():
                        drain(rs, rd, sem)
                        process(it, rs, rd, sem_o)

            for t in range(3):
                it = i0 + t
                rs, rd, sem, sem_o = bufs[t]
                if t == 0:
                    wait_out(rs, sem_o)
                else:
                    @pl.when(it < count)
                    def _():
                        wait_out(rs, sem_o)

                @pl.when(it + 3 < count)
                def _():
                    fire(it + 3, rs, rd, sem)

            return carry

        lax.fori_loop(0, n_triples, triple_body, 0)

    return k(gs, gd, edge_index)


_DNUM_T_RHS = (((1,), (1,)), ((), ()))


def _mlp_body(xt_ref, dt_ref, p_ref, il_ref, ir_ref, w1ct_ref, w1dt_ref,
              w2t_ref, b2_ref, g_ref, beta_ref, o_ref):
    xt = xt_ref[...]
    p = p_ref[...]
    e0 = lax.dot_general(il_ref[...], p, _DNUM_T_RHS,
                         preferred_element_type=jnp.float32)
    e1 = lax.dot_general(ir_ref[...], p, _DNUM_T_RHS,
                         preferred_element_type=jnp.float32)
    st = jnp.concatenate([e0, e1], axis=1)
    a = jnp.dot(w1ct_ref[...], xt, preferred_element_type=jnp.float32)
    a = a + jnp.dot(w1dt_ref[...], dt_ref[...],
                    preferred_element_type=jnp.float32)
    a = a + st
    h = a * jax.nn.sigmoid(a)
    h = jnp.dot(w2t_ref[...], h, preferred_element_type=jnp.float32)
    h = h + b2_ref[...]
    h = h * jax.nn.sigmoid(h)
    r = xt + h
    mu = jnp.mean(r, axis=0, keepdims=True)
    c = r - mu
    var = jnp.mean(c * c, axis=0, keepdims=True)
    o_ref[...] = c * lax.rsqrt(var + 1e-5) * g_ref[...] + beta_ref[...]


def _mlp(ef_t, d_t, s_packed, il, ir, w1c_t, w1d_t, w2_t, b2_c, g_c, beta_c):
    f, e = ef_t.shape
    r = d_t.shape[0]
    te = _TE
    assert e % te == 0
    grid = e // te
    return pl.pallas_call(
        _mlp_body,
        grid=(grid,),
        in_specs=[
            pl.BlockSpec((f, te), lambda i: (0, i)),
            pl.BlockSpec((r, te), lambda i: (0, i)),
            pl.BlockSpec((te // 2, 2 * f), lambda i: (i, 0)),
            pl.BlockSpec((f, 2 * f), lambda i: (0, 0)),
            pl.BlockSpec((f, 2 * f), lambda i: (0, 0)),
            pl.BlockSpec((f, f), lambda i: (0, 0)),
            pl.BlockSpec((f, r), lambda i: (0, 0)),
            pl.BlockSpec((f, f), lambda i: (0, 0)),
            pl.BlockSpec((f, 1), lambda i: (0, 0)),
            pl.BlockSpec((f, 1), lambda i: (0, 0)),
            pl.BlockSpec((f, 1), lambda i: (0, 0)),
        ],
        out_specs=pl.BlockSpec((f, te), lambda i: (0, i)),
        out_shape=jax.ShapeDtypeStruct((f, e), jnp.float32),
    )(ef_t, d_t, s_packed, il, ir, w1c_t, w1d_t, w2_t, b2_c, g_c, beta_c)


def kernel(node_scalars, edge_feats, d, W1, b1, W2, b2, ln_gamma, ln_beta,
           edge_index):
    n, s = node_scalars.shape
    e, f = edge_feats.shape
    w1a = W1[:s]
    w1b = W1[s:2 * s]
    w1c = W1[2 * s:2 * s + f]
    w1d = W1[2 * s + f:]

    gs, gd = _make_tables(node_scalars.T, w1a, w1b, b1.reshape(1, f))

    s_packed = _gather_add(gs, gd, edge_index)

    ident = jnp.eye(f, dtype=jnp.float32)
    zero = jnp.zeros((f, f), dtype=jnp.float32)
    out_t = _mlp(
        edge_feats.T,
        d.T,
        s_packed,
        jnp.concatenate([ident, zero], axis=1),
        jnp.concatenate([zero, ident], axis=1),
        w1c.T,
        w1d.T,
        W2.T,
        b2.reshape(f, 1),
        ln_gamma.reshape(f, 1),
        ln_beta.reshape(f, 1),
    )
    return out_t.T


# speedup vs baseline: 1.1081x; 1.1081x over previous
import functools

import jax
import jax.numpy as jnp
from jax import lax
from jax.experimental import pallas as pl
from jax.experimental.pallas import tpu as pltpu
from jax.experimental.pallas import tpu_sc as plsc


_DNUM_T_LHS = (((0,), (0,)), ((), ()))


def _tables_body(nst_ref, w1a_ref, w1b_ref, b1_ref, gs_ref, gd_ref):
    nst = nst_ref[...]
    gs = (
        lax.dot_general(nst, w1a_ref[...], _DNUM_T_LHS,
                        preferred_element_type=jnp.float32)
        + b1_ref[...]
    )
    gd = lax.dot_general(nst, w1b_ref[...], _DNUM_T_LHS,
                         preferred_element_type=jnp.float32)
    gs_ref[...] = gs
    gd_ref[...] = gd


def _make_tables(ns_t, w1a, w1b, b1_row):
    s, n = ns_t.shape
    f = w1a.shape[1]
    tn = 2048
    grid = (n + tn - 1) // tn
    return pl.pallas_call(
        _tables_body,
        grid=(grid,),
        in_specs=[
            pl.BlockSpec((s, tn), lambda i: (0, i)),
            pl.BlockSpec((s, f), lambda i: (0, 0)),
            pl.BlockSpec((s, f), lambda i: (0, 0)),
            pl.BlockSpec((1, f), lambda i: (0, 0)),
        ],
        out_specs=[
            pl.BlockSpec((tn, f), lambda i: (i, 0)),
            pl.BlockSpec((tn, f), lambda i: (i, 0)),
        ],
        out_shape=[
            jax.ShapeDtypeStruct((n, f), jnp.float32),
            jax.ShapeDtypeStruct((n, f), jnp.float32),
        ],
    )(ns_t, w1a, w1b, b1_row)


_C = 128
_TE = 6400


def _gather_add(gs, gd, edge_index):
    e = edge_index.shape[1]
    f = gs.shape[1]
    info = plsc.get_sparse_core_info()
    nw = info.num_cores * info.num_subcores
    n_chunks = e // _C
    assert e % _C == 0
    base_cnt = n_chunks // nw
    extra = n_chunks % nw
    maxc = base_cnt + (1 if extra else 0)
    mesh = plsc.VectorSubcoreMesh(core_axis_name="c", subcore_axis_name="s")

    @functools.partial(
        pl.kernel,
        mesh=mesh,
        compiler_params=pltpu.CompilerParams(use_tc_tiling_on_sc=False),
        out_type=jax.ShapeDtypeStruct((e // 2, 2 * f), jnp.float32),
        scratch_types=[
            pltpu.VMEM((maxc * _C,), jnp.int32),
            pltpu.VMEM((maxc * _C,), jnp.int32),
            pltpu.VMEM((_C, f), jnp.float32),
            pltpu.VMEM((_C, f), jnp.float32),
            pltpu.VMEM((_C, f), jnp.float32),
            pltpu.VMEM((_C, f), jnp.float32),
            pltpu.VMEM((_C, f), jnp.float32),
            pltpu.VMEM((_C, f), jnp.float32),
            pltpu.SemaphoreType.DMA,
            pltpu.SemaphoreType.DMA,
            pltpu.SemaphoreType.DMA,
        ],
    )
    def k(gs_hbm, gd_hbm, ei_hbm, out_hbm,
          ixs, ixd, rs_a, rd_a, rs_b, rd_b, rs_c, rd_c,
          sem_a, sem_b, sem_c):
        wid = lax.axis_index("s") * info.num_cores + lax.axis_index("c")
        count = base_cnt + jnp.where(wid < extra, 1, 0)
        base_edge = (wid * base_cnt + jnp.minimum(wid, extra)) * _C
        nbase = base_cnt * _C
        pltpu.sync_copy(ei_hbm.at[0, pl.ds(base_edge, nbase)],
                        ixs.at[pl.ds(0, nbase)])
        pltpu.sync_copy(ei_hbm.at[1, pl.ds(base_edge, nbase)],
                        ixd.at[pl.ds(0, nbase)])

        @pl.when(count > base_cnt)
        def _():
            pltpu.sync_copy(ei_hbm.at[0, pl.ds(base_edge + nbase, _C)],
                            ixs.at[pl.ds(nbase, _C)])
            pltpu.sync_copy(ei_hbm.at[1, pl.ds(base_edge + nbase, _C)],
                            ixd.at[pl.ds(nbase, _C)])

        def fire(i, rs, rd, sem):
            o = i * _C
            pltpu.async_copy(gs_hbm.at[ixs.at[pl.ds(o, _C)]], rs, sem)
            pltpu.async_copy(gd_hbm.at[ixd.at[pl.ds(o, _C)]], rd, sem)

        def drain(rs, rd, sem):
            pltpu.make_async_copy(gs_hbm.at[pl.ds(0, _C)], rs, sem).wait()
            pltpu.make_async_copy(gs_hbm.at[pl.ds(0, _C)], rd, sem).wait()

        def process(i, rs, rd):
            def row_body(rr, c2):
                for j in range(f // 16):
                    sl = pl.ds(j * 16, 16)
                    plsc.addupdate(rs.at[rr, sl], rd[rr, sl])
                return c2

            lax.fori_loop(0, _C, row_body, 0, unroll=2)
            eoff = base_edge + i * _C
            blk = eoff // _TE
            loc = eoff - blk * _TE
            half = loc // (_TE // 2)
            q0 = blk * (_TE // 2) + (loc - half * (_TE // 2))
            @pl.when(half == 0)
            def _():
                pltpu.sync_copy(rs, out_hbm.at[pl.ds(q0, _C), pl.ds(0, f)])

            @pl.when(half == 1)
            def _():
                pltpu.sync_copy(rs, out_hbm.at[pl.ds(q0, _C), pl.ds(f, f)])

        bufs = ((rs_a, rd_a, sem_a), (rs_b, rd_b, sem_b), (rs_c, rd_c, sem_c))
        fire(0, *bufs[0])

        @pl.when(1 < count)
        def _():
            fire(1, *bufs[1])

        @pl.when(2 < count)
        def _():
            fire(2, *bufs[2])

        n_triples = (count + 2) // 3

        def triple_body(j, carry):
            i0 = 3 * j
            for t in range(3):
                it = i0 + t
                rs, rd, sem = bufs[t]
                if t == 0:
                    drain(rs, rd, sem)
                    process(it, rs, rd)

                    @pl.when(it + 3 < count)
                    def _():
                        fire(it + 3, rs, rd, sem)
                else:
                    @pl.when(it < count)
                    def _():
                        drain(rs, rd, sem)
                        process(it, rs, rd)

                    @pl.when(it + 3 < count)
                    def _():
                        fire(it + 3, rs, rd, sem)

            return carry

        lax.fori_loop(0, n_triples, triple_body, 0)

    return k(gs, gd, edge_index)


_DNUM_T_RHS = (((1,), (1,)), ((), ()))


def _mlp_body(xt_ref, dt_ref, p_ref, il_ref, ir_ref, w1ct_ref, w1dt_ref,
              w2t_ref, b2_ref, g_ref, beta_ref, o_ref):
    xt = xt_ref[...]
    p = p_ref[...]
    e0 = lax.dot_general(il_ref[...], p, _DNUM_T_RHS,
                         preferred_element_type=jnp.float32)
    e1 = lax.dot_general(ir_ref[...], p, _DNUM_T_RHS,
                         preferred_element_type=jnp.float32)
    st = jnp.concatenate([e0, e1], axis=1)
    a = jnp.dot(w1ct_ref[...], xt, preferred_element_type=jnp.float32)
    a = a + jnp.dot(w1dt_ref[...], dt_ref[...],
                    preferred_element_type=jnp.float32)
    a = a + st
    h = a * jax.nn.sigmoid(a)
    h = jnp.dot(w2t_ref[...], h, preferred_element_type=jnp.float32)
    h = h + b2_ref[...]
    h = h * jax.nn.sigmoid(h)
    r = xt + h
    mu = jnp.mean(r, axis=0, keepdims=True)
    c = r - mu
    var = jnp.mean(c * c, axis=0, keepdims=True)
    o_ref[...] = c * lax.rsqrt(var + 1e-5) * g_ref[...] + beta_ref[...]


def _mlp(ef_t, d_t, s_packed, il, ir, w1c_t, w1d_t, w2_t, b2_c, g_c, beta_c):
    f, e = ef_t.shape
    r = d_t.shape[0]
    te = _TE
    assert e % te == 0
    grid = e // te
    return pl.pallas_call(
        _mlp_body,
        grid=(grid,),
        in_specs=[
            pl.BlockSpec((f, te), lambda i: (0, i)),
            pl.BlockSpec((r, te), lambda i: (0, i)),
            pl.BlockSpec((te // 2, 2 * f), lambda i: (i, 0)),
            pl.BlockSpec((f, 2 * f), lambda i: (0, 0)),
            pl.BlockSpec((f, 2 * f), lambda i: (0, 0)),
            pl.BlockSpec((f, f), lambda i: (0, 0)),
            pl.BlockSpec((f, r), lambda i: (0, 0)),
            pl.BlockSpec((f, f), lambda i: (0, 0)),
            pl.BlockSpec((f, 1), lambda i: (0, 0)),
            pl.BlockSpec((f, 1), lambda i: (0, 0)),
            pl.BlockSpec((f, 1), lambda i: (0, 0)),
        ],
        out_specs=pl.BlockSpec((f, te), lambda i: (0, i)),
        out_shape=jax.ShapeDtypeStruct((f, e), jnp.float32),
    )(ef_t, d_t, s_packed, il, ir, w1c_t, w1d_t, w2_t, b2_c, g_c, beta_c)


def kernel(node_scalars, edge_feats, d, W1, b1, W2, b2, ln_gamma, ln_beta,
           edge_index):
    n, s = node_scalars.shape
    e, f = edge_feats.shape
    w1a = W1[:s]
    w1b = W1[s:2 * s]
    w1c = W1[2 * s:2 * s + f]
    w1d = W1[2 * s + f:]

    gs, gd = _make_tables(node_scalars.T, w1a, w1b, b1.reshape(1, f))

    s_packed = _gather_add(gs, gd, edge_index)

    ident = jnp.eye(f, dtype=jnp.float32)
    zero = jnp.zeros((f, f), dtype=jnp.float32)
    out_t = _mlp(
        edge_feats.T,
        d.T,
        s_packed,
        jnp.concatenate([ident, zero], axis=1),
        jnp.concatenate([zero, ident], axis=1),
        w1c.T,
        w1d.T,
        W2.T,
        b2.reshape(f, 1),
        ln_gamma.reshape(f, 1),
        ln_beta.reshape(f, 1),
    )
    return out_t.T
